# baseline (speedup 1.0000x reference)
import jax
import jax.numpy as jnp
from jax import lax
from jax.experimental import pallas as pl
from jax.experimental.pallas import tpu as pltpu
from jax.experimental.pallas import tpu_sc as plsc

N_NODES = 10000
N_EDGES = 320000
IN_DIM = 128
NUM_HEADS = 8
OUT_DIM = 16
HD = NUM_HEADS * OUT_DIM
MZ = HD + 16

NC = 2
NS = 16
L = 16
NW = NC * NS
EDGES_PER_TILE = N_EDGES // NW
EB = 80
NBATCH = EDGES_PER_TILE // EB
CB = 80
NCHUNK = EDGES_PER_TILE // CB
NODE_PAD = 10240
NODE_ROWS_PER_TILE = NODE_PAD // NS


def _mm_body(x_ref, w_ref, o_ref):
    o_ref[...] = jnp.dot(x_ref[...], w_ref[...],
                         preferred_element_type=jnp.float32)


def _matmul(x, w, block_rows):
    m, k = x.shape
    n = w.shape[1]
    return pl.pallas_call(
        _mm_body,
        grid=(m // block_rows,),
        in_specs=[pl.BlockSpec((block_rows, k), lambda i: (i, 0)),
                  pl.BlockSpec((k, n), lambda i: (0, 0))],
        out_specs=pl.BlockSpec((block_rows, n), lambda i: (i, 0)),
        out_shape=jax.ShapeDtypeStruct((m, n), jnp.float32),
    )(x, w)


def _pass1_body(q_hbm, kv_hbm, p_hbm, src_hbm, dst_hbm,
                eout_hbm, mz_hbm,
                src_v, dst_v, kv_v, q_v, p_v, eout_v, mz_v,
                sem_g, sem_w):
    cid = lax.axis_index("c")
    sid = lax.axis_index("s")
    wid = sid * NC + cid

    zero16 = jnp.zeros((L,), jnp.float32)
    iota = lax.iota(jnp.int32, L)

    def _edge_compute(e_i, c):
        zacc = zero16
        for hh in range(NUM_HEADS):
            sl = pl.ds(hh * OUT_DIM, OUT_DIM)
            sc = kv_v[e_i, sl] * q_v[e_i, sl] * p_v[e_i, sl]
            eout_v[e_i, sl] = sc
            s = jnp.sum(sc)
            a = jnp.exp(jnp.clip(jnp.full((L,), s, jnp.float32), -5.0, 5.0))
            vx = kv_v[e_i, pl.ds(HD + hh * OUT_DIM, OUT_DIM)]
            mz_v[e_i, sl] = vx * a
            zacc = jnp.where(iota == hh, a, zacc)
        mz_v[e_i, pl.ds(HD, 16)] = zacc
        return c

    def _batch(b, c):
        base = wid * EDGES_PER_TILE + b * EB
        pltpu.sync_copy(src_hbm.at[pl.ds(base, EB)], src_v)
        pltpu.sync_copy(dst_hbm.at[pl.ds(base, EB)], dst_v)
        cp_kv = pltpu.async_copy(kv_hbm.at[src_v], kv_v, sem_g)
        cp_q = pltpu.async_copy(q_hbm.at[dst_v], q_v, sem_g)
        pltpu.sync_copy(p_hbm.at[pl.ds(base, EB)], p_v)
        cp_kv.wait()
        cp_q.wait()

        lax.fori_loop(0, EB, _edge_compute, 0)

        w1 = pltpu.async_copy(eout_v, eout_hbm.at[pl.ds(base, EB)], sem_w)
        w2 = pltpu.async_copy(mz_v, mz_hbm.at[pl.ds(base, EB)], sem_w)
        w1.wait()
        w2.wait()
        return c

    lax.fori_loop(0, NBATCH, _batch, 0)


_pass1_kernel = pl.kernel(
    _pass1_body,
    out_type=(jax.ShapeDtypeStruct((N_EDGES, HD), jnp.float32),
              jax.ShapeDtypeStruct((N_EDGES, MZ), jnp.float32)),
    mesh=plsc.VectorSubcoreMesh(core_axis_name="c", subcore_axis_name="s",
                                num_cores=NC, num_subcores=NS),
    compiler_params=pltpu.CompilerParams(use_tc_tiling_on_sc=False,
                                         needs_layout_passes=False),
    scratch_types=[
        pltpu.VMEM((EB,), jnp.int32),
        pltpu.VMEM((EB,), jnp.int32),
        pltpu.VMEM((EB, 2 * HD), jnp.float32),
        pltpu.VMEM((EB, HD), jnp.float32),
        pltpu.VMEM((EB, HD), jnp.float32),
        pltpu.VMEM((EB, HD), jnp.float32),
        pltpu.VMEM((EB, MZ), jnp.float32),
        pltpu.SemaphoreType.DMA,
        pltpu.SemaphoreType.DMA,
    ],
)


def _pass2_body(mz_hbm, dst_hbm, zmz_hbm,
                mz_parts_hbm,
                dst_v, mz_v, mz_sh, sem_g):
    cid = lax.axis_index("c")
    sid = lax.axis_index("s")
    wid = sid * NC + cid

    nbase = sid * NODE_ROWS_PER_TILE
    pltpu.sync_copy(zmz_hbm, mz_sh.at[pl.ds(nbase, NODE_ROWS_PER_TILE)])
    plsc.subcore_barrier()

    def _chunk(b, c):
        base = wid * EDGES_PER_TILE + b * CB
        pltpu.sync_copy(dst_hbm.at[pl.ds(base, CB)], dst_v)
        pltpu.async_copy(mz_hbm.at[pl.ds(base, CB)], mz_v, sem_g).wait()
        pltpu.sync_copy(mz_v, mz_sh.at[dst_v], add=True)
        return c

    lax.fori_loop(0, NCHUNK, _chunk, 0)
    plsc.subcore_barrier()

    pltpu.sync_copy(mz_sh.at[pl.ds(nbase, NODE_ROWS_PER_TILE)],
                    mz_parts_hbm.at[cid, pl.ds(nbase, NODE_ROWS_PER_TILE)])


_pass2_kernel = pl.kernel(
    _pass2_body,
    out_type=jax.ShapeDtypeStruct((NC, NODE_PAD, MZ), jnp.float32),
    mesh=plsc.VectorSubcoreMesh(core_axis_name="c", subcore_axis_name="s",
                                num_cores=NC, num_subcores=NS),
    compiler_params=pltpu.CompilerParams(use_tc_tiling_on_sc=False,
                                         needs_layout_passes=False),
    scratch_types=[
        pltpu.VMEM((CB,), jnp.int32),
        pltpu.VMEM((CB, MZ), jnp.float32),
        pltpu.VMEM_SHARED((NODE_PAD, MZ), jnp.float32),
        pltpu.SemaphoreType.DMA,
    ],
)


def _finalize_body(mz_ref, o_ref):
    mz = mz_ref[0] + mz_ref[1]
    wv = mz[:, 0:HD]
    z8 = mz[:, HD:HD + NUM_HEADS]
    row = lax.broadcasted_iota(jnp.int32, (NUM_HEADS, HD), 0)
    col = lax.broadcasted_iota(jnp.int32, (NUM_HEADS, HD), 1)
    expand = jnp.where(col // OUT_DIM == row, 1.0, 0.0)
    zrep = jnp.dot(z8, expand, preferred_element_type=jnp.float32)
    o_ref[...] = wv / (zrep + 1e-6)


def _finalize(mz_parts, block_rows=1024):
    return pl.pallas_call(
        _finalize_body,
        grid=(NODE_PAD // block_rows,),
        in_specs=[pl.BlockSpec((NC, block_rows, MZ), lambda i: (0, i, 0))],
        out_specs=pl.BlockSpec((block_rows, HD), lambda i: (i, 0)),
        out_shape=jax.ShapeDtypeStruct((NODE_PAD, HD), jnp.float32),
    )(mz_parts)


def kernel(h, e, edge_index, WQ, WK, WV, We):
    q_h = _matmul(h, WQ, 1000)
    kv = _matmul(h, jnp.concatenate([WK, WV], axis=1), 1000)
    p = _matmul(e, We * (1.0 / jnp.sqrt(jnp.float32(OUT_DIM))), 3200)

    src = edge_index[0]
    dst = edge_index[1]
    zmz = jnp.zeros((NODE_ROWS_PER_TILE, MZ), jnp.float32)

    e_out, mz = _pass1_kernel(q_h, kv, p, src, dst)
    mz_parts = _pass2_kernel(mz, dst, zmz)
    h_out = _finalize(mz_parts)

    return (h_out[:N_NODES].reshape(N_NODES, NUM_HEADS, OUT_DIM),
            e_out.reshape(N_EDGES, NUM_HEADS, OUT_DIM))


# speedup vs baseline: 1.5873x; 1.5873x over previous
import jax
import jax.numpy as jnp
from jax import lax
from jax.experimental import pallas as pl
from jax.experimental.pallas import tpu as pltpu
from jax.experimental.pallas import tpu_sc as plsc

N_NODES = 10000
N_EDGES = 320000
IN_DIM = 128
NUM_HEADS = 8
OUT_DIM = 16
HD = NUM_HEADS * OUT_DIM
MZ = HD + 16

NC = 2
NS = 16
L = 16
NW = NC * NS
EDGES_PER_TILE = N_EDGES // NW
EB = 80
NBATCH = EDGES_PER_TILE // EB
CB = 80
NCHUNK = EDGES_PER_TILE // CB
NODE_PAD = 10240
NODE_ROWS_PER_TILE = NODE_PAD // NS


def _mm_body(x_ref, w_ref, o_ref):
    o_ref[...] = jnp.dot(x_ref[...], w_ref[...],
                         preferred_element_type=jnp.float32)


def _matmul(x, w, block_rows):
    m, k = x.shape
    n = w.shape[1]
    return pl.pallas_call(
        _mm_body,
        grid=(m // block_rows,),
        in_specs=[pl.BlockSpec((block_rows, k), lambda i: (i, 0)),
                  pl.BlockSpec((k, n), lambda i: (0, 0))],
        out_specs=pl.BlockSpec((block_rows, n), lambda i: (i, 0)),
        out_shape=jax.ShapeDtypeStruct((m, n), jnp.float32),
    )(x, w)


def _pass1_body(q_hbm, kv_hbm, p_hbm, src_hbm, dst_hbm,
                eout_hbm, mz_hbm,
                src_v, dst_v, kv_v, q_v, p_v, eout_v, mz_v,
                sem_g, sem_w):
    cid = lax.axis_index("c")
    sid = lax.axis_index("s")
    wid = sid * NC + cid

    zero16 = jnp.zeros((L,), jnp.float32)
    iota = lax.iota(jnp.int32, L)

    IL = 4

    def _edge_quad(i, c):
        es = [i * IL + k for k in range(IL)]
        zaccs = [zero16] * IL
        for hh in range(NUM_HEADS):
            sl = pl.ds(hh * OUT_DIM, OUT_DIM)
            slv = pl.ds(HD + hh * OUT_DIM, OUT_DIM)
            scs = [kv_v[e, sl] * q_v[e, sl] * p_v[e, sl] for e in es]
            for k, e in enumerate(es):
                eout_v[e, sl] = scs[k]
            avs = [jnp.exp(jnp.clip(jnp.full((L,), jnp.sum(sc), jnp.float32),
                                    -5.0, 5.0)) for sc in scs]
            for k, e in enumerate(es):
                mz_v[e, sl] = kv_v[e, slv] * avs[k]
                zaccs[k] = jnp.where(iota == hh, avs[k], zaccs[k])
        for k, e in enumerate(es):
            mz_v[e, pl.ds(HD, 16)] = zaccs[k]
        return c

    def _batch(b, c):
        base = wid * EDGES_PER_TILE + b * EB
        pltpu.sync_copy(src_hbm.at[pl.ds(base, EB)], src_v)
        pltpu.sync_copy(dst_hbm.at[pl.ds(base, EB)], dst_v)
        cp_kv = pltpu.async_copy(kv_hbm.at[src_v], kv_v, sem_g)
        cp_q = pltpu.async_copy(q_hbm.at[dst_v], q_v, sem_g)
        pltpu.sync_copy(p_hbm.at[pl.ds(base, EB)], p_v)
        cp_kv.wait()
        cp_q.wait()

        lax.fori_loop(0, EB // IL, _edge_quad, 0)

        w1 = pltpu.async_copy(eout_v, eout_hbm.at[pl.ds(base, EB)], sem_w)
        w2 = pltpu.async_copy(mz_v, mz_hbm.at[pl.ds(base, EB)], sem_w)
        w1.wait()
        w2.wait()
        return c

    lax.fori_loop(0, NBATCH, _batch, 0)


_pass1_kernel = pl.kernel(
    _pass1_body,
    out_type=(jax.ShapeDtypeStruct((N_EDGES, HD), jnp.float32),
              jax.ShapeDtypeStruct((N_EDGES, MZ), jnp.float32)),
    mesh=plsc.VectorSubcoreMesh(core_axis_name="c", subcore_axis_name="s",
                                num_cores=NC, num_subcores=NS),
    compiler_params=pltpu.CompilerParams(use_tc_tiling_on_sc=False,
                                         needs_layout_passes=False),
    scratch_types=[
        pltpu.VMEM((EB,), jnp.int32),
        pltpu.VMEM((EB,), jnp.int32),
        pltpu.VMEM((EB, 2 * HD), jnp.float32),
        pltpu.VMEM((EB, HD), jnp.float32),
        pltpu.VMEM((EB, HD), jnp.float32),
        pltpu.VMEM((EB, HD), jnp.float32),
        pltpu.VMEM((EB, MZ), jnp.float32),
        pltpu.SemaphoreType.DMA,
        pltpu.SemaphoreType.DMA,
    ],
)


def _pass2_body(mz_hbm, dst_hbm, zmz_hbm,
                mz_parts_hbm,
                dst_v, mz_v, mz_sh, sem_g):
    cid = lax.axis_index("c")
    sid = lax.axis_index("s")
    wid = sid * NC + cid

    nbase = sid * NODE_ROWS_PER_TILE
    pltpu.sync_copy(zmz_hbm, mz_sh.at[pl.ds(nbase, NODE_ROWS_PER_TILE)])
    plsc.subcore_barrier()

    def _chunk(b, c):
        base = wid * EDGES_PER_TILE + b * CB
        pltpu.sync_copy(dst_hbm.at[pl.ds(base, CB)], dst_v)
        pltpu.async_copy(mz_hbm.at[pl.ds(base, CB)], mz_v, sem_g).wait()
        pltpu.sync_copy(mz_v, mz_sh.at[dst_v], add=True)
        return c

    lax.fori_loop(0, NCHUNK, _chunk, 0)
    plsc.subcore_barrier()

    pltpu.sync_copy(mz_sh.at[pl.ds(nbase, NODE_ROWS_PER_TILE)],
                    mz_parts_hbm.at[cid, pl.ds(nbase, NODE_ROWS_PER_TILE)])


_pass2_kernel = pl.kernel(
    _pass2_body,
    out_type=jax.ShapeDtypeStruct((NC, NODE_PAD, MZ), jnp.float32),
    mesh=plsc.VectorSubcoreMesh(core_axis_name="c", subcore_axis_name="s",
                                num_cores=NC, num_subcores=NS),
    compiler_params=pltpu.CompilerParams(use_tc_tiling_on_sc=False,
                                         needs_layout_passes=False),
    scratch_types=[
        pltpu.VMEM((CB,), jnp.int32),
        pltpu.VMEM((CB, MZ), jnp.float32),
        pltpu.VMEM_SHARED((NODE_PAD, MZ), jnp.float32),
        pltpu.SemaphoreType.DMA,
    ],
)


def _finalize_body(mz_ref, o_ref):
    mz = mz_ref[0] + mz_ref[1]
    wv = mz[:, 0:HD]
    z8 = mz[:, HD:HD + NUM_HEADS]
    row = lax.broadcasted_iota(jnp.int32, (NUM_HEADS, HD), 0)
    col = lax.broadcasted_iota(jnp.int32, (NUM_HEADS, HD), 1)
    expand = jnp.where(col // OUT_DIM == row, 1.0, 0.0)
    zrep = jnp.dot(z8, expand, preferred_element_type=jnp.float32)
    o_ref[...] = wv / (zrep + 1e-6)


def _finalize(mz_parts, block_rows=1024):
    return pl.pallas_call(
        _finalize_body,
        grid=(NODE_PAD // block_rows,),
        in_specs=[pl.BlockSpec((NC, block_rows, MZ), lambda i: (0, i, 0))],
        out_specs=pl.BlockSpec((block_rows, HD), lambda i: (i, 0)),
        out_shape=jax.ShapeDtypeStruct((NODE_PAD, HD), jnp.float32),
    )(mz_parts)


def kernel(h, e, edge_index, WQ, WK, WV, We):
    q_h = _matmul(h, WQ, 1000)
    kv = _matmul(h, jnp.concatenate([WK, WV], axis=1), 1000)
    p = _matmul(e, We * (1.0 / jnp.sqrt(jnp.float32(OUT_DIM))), 3200)

    src = edge_index[0]
    dst = edge_index[1]
    zmz = jnp.zeros((NODE_ROWS_PER_TILE, MZ), jnp.float32)

    e_out, mz = _pass1_kernel(q_h, kv, p, src, dst)
    mz_parts = _pass2_kernel(mz, dst, zmz)
    h_out = _finalize(mz_parts)

    return (h_out[:N_NODES].reshape(N_NODES, NUM_HEADS, OUT_DIM),
            e_out.reshape(N_EDGES, NUM_HEADS, OUT_DIM))


# speedup vs baseline: 1.5925x; 1.0032x over previous
import jax
import jax.numpy as jnp
from jax import lax
from jax.experimental import pallas as pl
from jax.experimental.pallas import tpu as pltpu
from jax.experimental.pallas import tpu_sc as plsc

N_NODES = 10000
N_EDGES = 320000
IN_DIM = 128
NUM_HEADS = 8
OUT_DIM = 16
HD = NUM_HEADS * OUT_DIM
MZ = HD + 16

NC = 2
NS = 16
L = 16
NW = NC * NS
EDGES_PER_TILE = N_EDGES // NW
EB = 80
NBATCH = EDGES_PER_TILE // EB
CB = 80
NCHUNK = EDGES_PER_TILE // CB
NODE_PAD = 10240
NODE_ROWS_PER_TILE = NODE_PAD // NS


def _mm_body(x_ref, w_ref, o_ref):
    o_ref[...] = jnp.dot(x_ref[...], w_ref[...],
                         preferred_element_type=jnp.float32)


def _matmul(x, w, block_rows):
    m, k = x.shape
    n = w.shape[1]
    return pl.pallas_call(
        _mm_body,
        grid=(m // block_rows,),
        in_specs=[pl.BlockSpec((block_rows, k), lambda i: (i, 0)),
                  pl.BlockSpec((k, n), lambda i: (0, 0))],
        out_specs=pl.BlockSpec((block_rows, n), lambda i: (i, 0)),
        out_shape=jax.ShapeDtypeStruct((m, n), jnp.float32),
    )(x, w)


def _pass1_body(q_hbm, kv_hbm, p_hbm, ei_hbm,
                eout_hbm, mz_hbm,
                src_v, dst_v, kv_v, q_v, p_v, eout_v, mz_v,
                sem_g, sem_w):
    cid = lax.axis_index("c")
    sid = lax.axis_index("s")
    wid = sid * NC + cid

    zero16 = jnp.zeros((L,), jnp.float32)
    iota = lax.iota(jnp.int32, L)

    IL = 4

    def _edge_quad(i, c):
        es = [i * IL + k for k in range(IL)]
        zaccs = [zero16] * IL
        for hh in range(NUM_HEADS):
            sl = pl.ds(hh * OUT_DIM, OUT_DIM)
            slv = pl.ds(HD + hh * OUT_DIM, OUT_DIM)
            scs = [kv_v[e, sl] * q_v[e, sl] * p_v[e, sl] for e in es]
            for k, e in enumerate(es):
                eout_v[e, sl] = scs[k]
            avs = [jnp.exp(jnp.clip(jnp.full((L,), jnp.sum(sc), jnp.float32),
                                    -5.0, 5.0)) for sc in scs]
            for k, e in enumerate(es):
                mz_v[e, sl] = kv_v[e, slv] * avs[k]
                zaccs[k] = jnp.where(iota == hh, avs[k], zaccs[k])
        for k, e in enumerate(es):
            mz_v[e, pl.ds(HD, 16)] = zaccs[k]
        return c

    def _batch(b, c):
        base = wid * EDGES_PER_TILE + b * EB
        pltpu.sync_copy(ei_hbm.at[0, pl.ds(base, EB)], src_v)
        pltpu.sync_copy(ei_hbm.at[1, pl.ds(base, EB)], dst_v)
        cp_kv = pltpu.async_copy(kv_hbm.at[src_v], kv_v, sem_g)
        cp_q = pltpu.async_copy(q_hbm.at[dst_v], q_v, sem_g)
        pltpu.sync_copy(p_hbm.at[pl.ds(base, EB)], p_v)
        cp_kv.wait()
        cp_q.wait()

        lax.fori_loop(0, EB // IL, _edge_quad, 0)

        w1 = pltpu.async_copy(eout_v, eout_hbm.at[pl.ds(base, EB)], sem_w)
        w2 = pltpu.async_copy(mz_v, mz_hbm.at[pl.ds(base, EB)], sem_w)
        w1.wait()
        w2.wait()
        return c

    lax.fori_loop(0, NBATCH, _batch, 0)


_pass1_kernel = pl.kernel(
    _pass1_body,
    out_type=(jax.ShapeDtypeStruct((N_EDGES, HD), jnp.float32),
              jax.ShapeDtypeStruct((N_EDGES, MZ), jnp.float32)),
    mesh=plsc.VectorSubcoreMesh(core_axis_name="c", subcore_axis_name="s",
                                num_cores=NC, num_subcores=NS),
    compiler_params=pltpu.CompilerParams(use_tc_tiling_on_sc=False,
                                         needs_layout_passes=False),
    scratch_types=[
        pltpu.VMEM((EB,), jnp.int32),
        pltpu.VMEM((EB,), jnp.int32),
        pltpu.VMEM((EB, 2 * HD), jnp.float32),
        pltpu.VMEM((EB, HD), jnp.float32),
        pltpu.VMEM((EB, HD), jnp.float32),
        pltpu.VMEM((EB, HD), jnp.float32),
        pltpu.VMEM((EB, MZ), jnp.float32),
        pltpu.SemaphoreType.DMA,
        pltpu.SemaphoreType.DMA,
    ],
)


def _pass2_body(mz_hbm, ei_hbm, zmz_hbm,
                mz_parts_hbm,
                dst_v, mz_v, mz_sh, sem_g):
    cid = lax.axis_index("c")
    sid = lax.axis_index("s")
    wid = sid * NC + cid

    nbase = sid * NODE_ROWS_PER_TILE
    pltpu.sync_copy(zmz_hbm, mz_sh.at[pl.ds(nbase, NODE_ROWS_PER_TILE)])
    plsc.subcore_barrier()

    def _chunk(b, c):
        base = wid * EDGES_PER_TILE + b * CB
        pltpu.sync_copy(ei_hbm.at[1, pl.ds(base, CB)], dst_v)
        pltpu.async_copy(mz_hbm.at[pl.ds(base, CB)], mz_v, sem_g).wait()
        pltpu.sync_copy(mz_v, mz_sh.at[dst_v], add=True)
        return c

    lax.fori_loop(0, NCHUNK, _chunk, 0)
    plsc.subcore_barrier()

    pltpu.sync_copy(mz_sh.at[pl.ds(nbase, NODE_ROWS_PER_TILE)],
                    mz_parts_hbm.at[cid, pl.ds(nbase, NODE_ROWS_PER_TILE)])


_pass2_kernel = pl.kernel(
    _pass2_body,
    out_type=jax.ShapeDtypeStruct((NC, NODE_PAD, MZ), jnp.float32),
    mesh=plsc.VectorSubcoreMesh(core_axis_name="c", subcore_axis_name="s",
                                num_cores=NC, num_subcores=NS),
    compiler_params=pltpu.CompilerParams(use_tc_tiling_on_sc=False,
                                         needs_layout_passes=False),
    scratch_types=[
        pltpu.VMEM((CB,), jnp.int32),
        pltpu.VMEM((CB, MZ), jnp.float32),
        pltpu.VMEM_SHARED((NODE_PAD, MZ), jnp.float32),
        pltpu.SemaphoreType.DMA,
    ],
)


def _finalize_body(mz_ref, o_ref):
    mz = mz_ref[0] + mz_ref[1]
    wv = mz[:, 0:HD]
    z8 = mz[:, HD:HD + NUM_HEADS]
    row = lax.broadcasted_iota(jnp.int32, (NUM_HEADS, HD), 0)
    col = lax.broadcasted_iota(jnp.int32, (NUM_HEADS, HD), 1)
    expand = jnp.where(col // OUT_DIM == row, 1.0, 0.0)
    zrep = jnp.dot(z8, expand, preferred_element_type=jnp.float32)
    o_ref[...] = wv / (zrep + 1e-6)


def _finalize(mz_parts, block_rows=1024):
    return pl.pallas_call(
        _finalize_body,
        grid=(NODE_PAD // block_rows,),
        in_specs=[pl.BlockSpec((NC, block_rows, MZ), lambda i: (0, i, 0))],
        out_specs=pl.BlockSpec((block_rows, HD), lambda i: (i, 0)),
        out_shape=jax.ShapeDtypeStruct((NODE_PAD, HD), jnp.float32),
    )(mz_parts)


def kernel(h, e, edge_index, WQ, WK, WV, We):
    q_h = _matmul(h, WQ, 1000)
    kv = _matmul(h, jnp.concatenate([WK, WV], axis=1), 1000)
    p = _matmul(e, We * (1.0 / jnp.sqrt(jnp.float32(OUT_DIM))), 3200)

    zmz = jnp.zeros((NODE_ROWS_PER_TILE, MZ), jnp.float32)

    e_out, mz = _pass1_kernel(q_h, kv, p, edge_index)
    mz_parts = _pass2_kernel(mz, edge_index, zmz)
    h_out = _finalize(mz_parts)

    return (h_out[:N_NODES].reshape(N_NODES, NUM_HEADS, OUT_DIM),
            e_out.reshape(N_EDGES, NUM_HEADS, OUT_DIM))


# speedup vs baseline: 2.0246x; 1.2714x over previous
import jax
import jax.numpy as jnp
from jax import lax
from jax.experimental import pallas as pl
from jax.experimental.pallas import tpu as pltpu
from jax.experimental.pallas import tpu_sc as plsc

N_NODES = 10000
N_EDGES = 320000
IN_DIM = 128
NUM_HEADS = 8
OUT_DIM = 16
HD = NUM_HEADS * OUT_DIM
MZ = HD + 16

NC = 2
NS = 16
L = 16
NW = NC * NS
EDGES_PER_TILE = N_EDGES // NW
EB = 80
NBATCH = EDGES_PER_TILE // EB
CB = 80
NCHUNK = EDGES_PER_TILE // CB
NODE_PAD = 10240
NODE_ROWS_PER_TILE = NODE_PAD // NS


def _mm_body(x_ref, w_ref, o_ref):
    o_ref[...] = jnp.dot(x_ref[...], w_ref[...],
                         preferred_element_type=jnp.float32)


def _matmul(x, w, block_rows):
    m, k = x.shape
    n = w.shape[1]
    return pl.pallas_call(
        _mm_body,
        grid=(m // block_rows,),
        in_specs=[pl.BlockSpec((block_rows, k), lambda i: (i, 0)),
                  pl.BlockSpec((k, n), lambda i: (0, 0))],
        out_specs=pl.BlockSpec((block_rows, n), lambda i: (i, 0)),
        out_shape=jax.ShapeDtypeStruct((m, n), jnp.float32),
    )(x, w)


def _pass1_body(q_hbm, kv_hbm, p_hbm, ei_hbm,
                eout_hbm, mz_hbm,
                src0, dst0, kv0, q0, p0, eout0, mz0,
                src1, dst1, kv1, q1, p1, eout1, mz1,
                sg0, sg1, sw0, sw1):
    cid = lax.axis_index("c")
    sid = lax.axis_index("s")
    wid = sid * NC + cid

    zero16 = jnp.zeros((L,), jnp.float32)
    iota = lax.iota(jnp.int32, L)

    set0 = (src0, dst0, kv0, q0, p0, sg0)
    set1 = (src1, dst1, kv1, q1, p1, sg1)
    out0 = (eout0, mz0, sw0)
    out1 = (eout1, mz1, sw1)

    def issue_gathers(b, bufs):
        src_v, dst_v, kv_v, q_v, p_v, sg = bufs
        base = wid * EDGES_PER_TILE + b * EB
        pltpu.sync_copy(ei_hbm.at[0, pl.ds(base, EB)], src_v)
        pltpu.sync_copy(ei_hbm.at[1, pl.ds(base, EB)], dst_v)
        pltpu.async_copy(kv_hbm.at[src_v], kv_v, sg)
        pltpu.async_copy(q_hbm.at[dst_v], q_v, sg)
        pltpu.async_copy(p_hbm.at[pl.ds(base, EB)], p_v, sg)

    def wait_gathers(bufs):
        _, _, kv_v, q_v, p_v, sg = bufs
        pltpu.make_async_copy(kv_hbm.at[pl.ds(0, EB)], kv_v, sg).wait()
        pltpu.make_async_copy(q_hbm.at[pl.ds(0, EB)], q_v, sg).wait()
        pltpu.make_async_copy(p_hbm.at[pl.ds(0, EB)], p_v, sg).wait()

    def issue_wb(b, outs):
        eout_v, mz_v, sw = outs
        base = wid * EDGES_PER_TILE + b * EB
        pltpu.async_copy(eout_v, eout_hbm.at[pl.ds(base, EB)], sw)
        pltpu.async_copy(mz_v, mz_hbm.at[pl.ds(base, EB)], sw)

    def wait_wb(outs):
        eout_v, mz_v, sw = outs
        pltpu.make_async_copy(eout_v, eout_hbm.at[pl.ds(0, EB)], sw).wait()
        pltpu.make_async_copy(mz_v, mz_hbm.at[pl.ds(0, EB)], sw).wait()

    IL = 4

    def compute(bufs, outs):
        _, _, kv_v, q_v, p_v, _ = bufs
        eout_v, mz_v, _ = outs

        def _edge_quad(i, c):
            es = [i * IL + k for k in range(IL)]
            zaccs = [zero16] * IL
            for hh in range(NUM_HEADS):
                sl = pl.ds(hh * OUT_DIM, OUT_DIM)
                slv = pl.ds(HD + hh * OUT_DIM, OUT_DIM)
                scs = [kv_v[e, sl] * q_v[e, sl] * p_v[e, sl] for e in es]
                for k, e in enumerate(es):
                    eout_v[e, sl] = scs[k]
                avs = [jnp.exp(jnp.clip(jnp.full((L,), jnp.sum(sc),
                                                 jnp.float32),
                                        -5.0, 5.0)) for sc in scs]
                for k, e in enumerate(es):
                    mz_v[e, sl] = kv_v[e, slv] * avs[k]
                    zaccs[k] = jnp.where(iota == hh, avs[k], zaccs[k])
            for k, e in enumerate(es):
                mz_v[e, pl.ds(HD, 16)] = zaccs[k]
            return c

        lax.fori_loop(0, EB // IL, _edge_quad, 0)

    issue_gathers(0, set0)

    def _pair(i, c):
        b0 = i * 2

        issue_gathers(b0 + 1, set1)
        wait_gathers(set0)
        pl.when(i > 0)(lambda: wait_wb(out0))
        compute(set0, out0)
        issue_wb(b0, out0)

        issue_gathers(b0 + 2, set0)
        wait_gathers(set1)
        pl.when(i > 0)(lambda: wait_wb(out1))
        compute(set1, out1)
        issue_wb(b0 + 1, out1)
        return c

    lax.fori_loop(0, (NBATCH - 1) // 2, _pair, 0)

    wait_gathers(set0)
    wait_wb(out0)
    compute(set0, out0)
    issue_wb(NBATCH - 1, out0)
    wait_wb(out0)
    wait_wb(out1)


_pass1_kernel = pl.kernel(
    _pass1_body,
    out_type=(jax.ShapeDtypeStruct((N_EDGES, HD), jnp.float32),
              jax.ShapeDtypeStruct((N_EDGES, MZ), jnp.float32)),
    mesh=plsc.VectorSubcoreMesh(core_axis_name="c", subcore_axis_name="s",
                                num_cores=NC, num_subcores=NS),
    compiler_params=pltpu.CompilerParams(use_tc_tiling_on_sc=False,
                                         needs_layout_passes=False),
    scratch_types=[
        pltpu.VMEM((EB,), jnp.int32),
        pltpu.VMEM((EB,), jnp.int32),
        pltpu.VMEM((EB, 2 * HD), jnp.float32),
        pltpu.VMEM((EB, HD), jnp.float32),
        pltpu.VMEM((EB, HD), jnp.float32),
        pltpu.VMEM((EB, HD), jnp.float32),
        pltpu.VMEM((EB, MZ), jnp.float32),
        pltpu.VMEM((EB,), jnp.int32),
        pltpu.VMEM((EB,), jnp.int32),
        pltpu.VMEM((EB, 2 * HD), jnp.float32),
        pltpu.VMEM((EB, HD), jnp.float32),
        pltpu.VMEM((EB, HD), jnp.float32),
        pltpu.VMEM((EB, HD), jnp.float32),
        pltpu.VMEM((EB, MZ), jnp.float32),
        pltpu.SemaphoreType.DMA,
        pltpu.SemaphoreType.DMA,
        pltpu.SemaphoreType.DMA,
        pltpu.SemaphoreType.DMA,
    ],
)


def _pass2_body(mz_hbm, ei_hbm, zmz_hbm,
                mz_parts_hbm,
                dst_v, mz_v, mz_sh, sem_g):
    cid = lax.axis_index("c")
    sid = lax.axis_index("s")
    wid = sid * NC + cid

    nbase = sid * NODE_ROWS_PER_TILE
    pltpu.sync_copy(zmz_hbm, mz_sh.at[pl.ds(nbase, NODE_ROWS_PER_TILE)])
    plsc.subcore_barrier()

    def _chunk(b, c):
        base = wid * EDGES_PER_TILE + b * CB
        pltpu.sync_copy(ei_hbm.at[1, pl.ds(base, CB)], dst_v)
        pltpu.async_copy(mz_hbm.at[pl.ds(base, CB)], mz_v, sem_g).wait()
        pltpu.sync_copy(mz_v, mz_sh.at[dst_v], add=True)
        return c

    lax.fori_loop(0, NCHUNK, _chunk, 0)
    plsc.subcore_barrier()

    pltpu.sync_copy(mz_sh.at[pl.ds(nbase, NODE_ROWS_PER_TILE)],
                    mz_parts_hbm.at[cid, pl.ds(nbase, NODE_ROWS_PER_TILE)])


_pass2_kernel = pl.kernel(
    _pass2_body,
    out_type=jax.ShapeDtypeStruct((NC, NODE_PAD, MZ), jnp.float32),
    mesh=plsc.VectorSubcoreMesh(core_axis_name="c", subcore_axis_name="s",
                                num_cores=NC, num_subcores=NS),
    compiler_params=pltpu.CompilerParams(use_tc_tiling_on_sc=False,
                                         needs_layout_passes=False),
    scratch_types=[
        pltpu.VMEM((CB,), jnp.int32),
        pltpu.VMEM((CB, MZ), jnp.float32),
        pltpu.VMEM_SHARED((NODE_PAD, MZ), jnp.float32),
        pltpu.SemaphoreType.DMA,
    ],
)


def _finalize_body(mz_ref, o_ref):
    mz = mz_ref[0] + mz_ref[1]
    wv = mz[:, 0:HD]
    z8 = mz[:, HD:HD + NUM_HEADS]
    row = lax.broadcasted_iota(jnp.int32, (NUM_HEADS, HD), 0)
    col = lax.broadcasted_iota(jnp.int32, (NUM_HEADS, HD), 1)
    expand = jnp.where(col // OUT_DIM == row, 1.0, 0.0)
    zrep = jnp.dot(z8, expand, preferred_element_type=jnp.float32)
    o_ref[...] = wv / (zrep + 1e-6)


def _finalize(mz_parts, block_rows=1024):
    return pl.pallas_call(
        _finalize_body,
        grid=(NODE_PAD // block_rows,),
        in_specs=[pl.BlockSpec((NC, block_rows, MZ), lambda i: (0, i, 0))],
        out_specs=pl.BlockSpec((block_rows, HD), lambda i: (i, 0)),
        out_shape=jax.ShapeDtypeStruct((NODE_PAD, HD), jnp.float32),
    )(mz_parts)


def kernel(h, e, edge_index, WQ, WK, WV, We):
    q_h = _matmul(h, WQ, 1000)
    kv = _matmul(h, jnp.concatenate([WK, WV], axis=1), 1000)
    p = _matmul(e, We * (1.0 / jnp.sqrt(jnp.float32(OUT_DIM))), 3200)

    zmz = jnp.zeros((NODE_ROWS_PER_TILE, MZ), jnp.float32)

    e_out, mz = _pass1_kernel(q_h, kv, p, edge_index)
    mz_parts = _pass2_kernel(mz, edge_index, zmz)
    h_out = _finalize(mz_parts)

    return (h_out[:N_NODES].reshape(N_NODES, NUM_HEADS, OUT_DIM),
            e_out.reshape(N_EDGES, NUM_HEADS, OUT_DIM))


# speedup vs baseline: 2.2015x; 1.0873x over previous
import jax
import jax.numpy as jnp
from jax import lax
from jax.experimental import pallas as pl
from jax.experimental.pallas import tpu as pltpu
from jax.experimental.pallas import tpu_sc as plsc

N_NODES = 10000
N_EDGES = 320000
IN_DIM = 128
NUM_HEADS = 8
OUT_DIM = 16
HD = NUM_HEADS * OUT_DIM
MZ = HD + 16

NC = 2
NS = 16
L = 16
NW = NC * NS
EDGES_PER_TILE = N_EDGES // NW
EB = 80
NBATCH = EDGES_PER_TILE // EB
CB = 80
NCHUNK = EDGES_PER_TILE // CB
NODE_PAD = 10240
NODE_ROWS_PER_TILE = NODE_PAD // NS


def _mm_body(x_ref, w_ref, o_ref):
    o_ref[...] = jnp.dot(x_ref[...], w_ref[...],
                         preferred_element_type=jnp.float32)


def _matmul(x, w, block_rows):
    m, k = x.shape
    n = w.shape[1]
    return pl.pallas_call(
        _mm_body,
        grid=(m // block_rows,),
        in_specs=[pl.BlockSpec((block_rows, k), lambda i: (i, 0)),
                  pl.BlockSpec((k, n), lambda i: (0, 0))],
        out_specs=pl.BlockSpec((block_rows, n), lambda i: (i, 0)),
        out_shape=jax.ShapeDtypeStruct((m, n), jnp.float32),
    )(x, w)


def _pass1_body(q_hbm, kv_hbm, p_hbm, ei_hbm,
                eout_hbm, mz_hbm,
                src0, dst0, kv0, q0, p0, eout0, mz0,
                src1, dst1, kv1, q1, p1, eout1, mz1,
                sg0, sg1, sw0, sw1):
    cid = lax.axis_index("c")
    sid = lax.axis_index("s")
    wid = sid * NC + cid

    zero16 = jnp.zeros((L,), jnp.float32)
    iota = lax.iota(jnp.int32, L)

    set0 = (src0, dst0, kv0, q0, p0, sg0)
    set1 = (src1, dst1, kv1, q1, p1, sg1)
    out0 = (eout0, mz0, sw0)
    out1 = (eout1, mz1, sw1)

    def issue_gathers(b, bufs):
        src_v, dst_v, kv_v, q_v, p_v, sg = bufs
        base = wid * EDGES_PER_TILE + b * EB
        pltpu.sync_copy(ei_hbm.at[0, pl.ds(base, EB)], src_v)
        pltpu.sync_copy(ei_hbm.at[1, pl.ds(base, EB)], dst_v)
        pltpu.async_copy(kv_hbm.at[src_v], kv_v, sg)
        pltpu.async_copy(q_hbm.at[dst_v], q_v, sg)
        pltpu.async_copy(p_hbm.at[pl.ds(base, EB)], p_v, sg)

    def wait_gathers(bufs):
        _, _, kv_v, q_v, p_v, sg = bufs
        pltpu.make_async_copy(kv_hbm.at[pl.ds(0, EB)], kv_v, sg).wait()
        pltpu.make_async_copy(q_hbm.at[pl.ds(0, EB)], q_v, sg).wait()
        pltpu.make_async_copy(p_hbm.at[pl.ds(0, EB)], p_v, sg).wait()

    def issue_wb(b, outs):
        eout_v, mz_v, sw = outs
        base = wid * EDGES_PER_TILE + b * EB
        pltpu.async_copy(eout_v, eout_hbm.at[pl.ds(base, EB)], sw)
        pltpu.async_copy(mz_v, mz_hbm.at[pl.ds(base, EB)], sw)

    def wait_wb(outs):
        eout_v, mz_v, sw = outs
        pltpu.make_async_copy(eout_v, eout_hbm.at[pl.ds(0, EB)], sw).wait()
        pltpu.make_async_copy(mz_v, mz_hbm.at[pl.ds(0, EB)], sw).wait()

    IL = 4

    def compute(bufs, outs):
        _, _, kv_v, q_v, p_v, _ = bufs
        eout_v, mz_v, _ = outs

        def _edge_quad(i, c):
            es = [i * IL + k for k in range(IL)]
            zaccs = [zero16] * IL
            for hh in range(NUM_HEADS):
                sl = pl.ds(hh * OUT_DIM, OUT_DIM)
                slv = pl.ds(HD + hh * OUT_DIM, OUT_DIM)
                scs = [kv_v[e, sl] * q_v[e, sl] * p_v[e, sl] for e in es]
                for k, e in enumerate(es):
                    eout_v[e, sl] = scs[k]
                avs = [jnp.exp(jnp.clip(jnp.full((L,), jnp.sum(sc),
                                                 jnp.float32),
                                        -5.0, 5.0)) for sc in scs]
                for k, e in enumerate(es):
                    mz_v[e, sl] = kv_v[e, slv] * avs[k]
                    zaccs[k] = jnp.where(iota == hh, avs[k], zaccs[k])
            for k, e in enumerate(es):
                mz_v[e, pl.ds(HD, 16)] = zaccs[k]
            return c

        lax.fori_loop(0, EB // IL, _edge_quad, 0)

    issue_gathers(0, set0)

    def _pair(i, c):
        b0 = i * 2

        issue_gathers(b0 + 1, set1)
        wait_gathers(set0)
        pl.when(i > 0)(lambda: wait_wb(out0))
        compute(set0, out0)
        issue_wb(b0, out0)

        issue_gathers(b0 + 2, set0)
        wait_gathers(set1)
        pl.when(i > 0)(lambda: wait_wb(out1))
        compute(set1, out1)
        issue_wb(b0 + 1, out1)
        return c

    lax.fori_loop(0, (NBATCH - 1) // 2, _pair, 0)

    wait_gathers(set0)
    wait_wb(out0)
    compute(set0, out0)
    issue_wb(NBATCH - 1, out0)
    wait_wb(out0)
    wait_wb(out1)


_pass1_kernel = pl.kernel(
    _pass1_body,
    out_type=(jax.ShapeDtypeStruct((N_EDGES, HD), jnp.float32),
              jax.ShapeDtypeStruct((N_EDGES, MZ), jnp.float32)),
    mesh=plsc.VectorSubcoreMesh(core_axis_name="c", subcore_axis_name="s",
                                num_cores=NC, num_subcores=NS),
    compiler_params=pltpu.CompilerParams(use_tc_tiling_on_sc=False,
                                         needs_layout_passes=False),
    scratch_types=[
        pltpu.VMEM((EB,), jnp.int32),
        pltpu.VMEM((EB,), jnp.int32),
        pltpu.VMEM((EB, 2 * HD), jnp.float32),
        pltpu.VMEM((EB, HD), jnp.float32),
        pltpu.VMEM((EB, HD), jnp.float32),
        pltpu.VMEM((EB, HD), jnp.float32),
        pltpu.VMEM((EB, MZ), jnp.float32),
        pltpu.VMEM((EB,), jnp.int32),
        pltpu.VMEM((EB,), jnp.int32),
        pltpu.VMEM((EB, 2 * HD), jnp.float32),
        pltpu.VMEM((EB, HD), jnp.float32),
        pltpu.VMEM((EB, HD), jnp.float32),
        pltpu.VMEM((EB, HD), jnp.float32),
        pltpu.VMEM((EB, MZ), jnp.float32),
        pltpu.SemaphoreType.DMA,
        pltpu.SemaphoreType.DMA,
        pltpu.SemaphoreType.DMA,
        pltpu.SemaphoreType.DMA,
    ],
)


def _pass2_body(mz_hbm, ei_hbm, zmz_hbm,
                mz_parts_hbm,
                dst0, mz0, dst1, mz1, mz_sh, sg0, sg1):
    cid = lax.axis_index("c")
    sid = lax.axis_index("s")
    wid = sid * NC + cid

    nbase = sid * NODE_ROWS_PER_TILE
    pltpu.sync_copy(zmz_hbm, mz_sh.at[pl.ds(nbase, NODE_ROWS_PER_TILE)])
    plsc.subcore_barrier()

    s0 = (dst0, mz0, sg0)
    s1 = (dst1, mz1, sg1)

    def issue_load(b, bufs):
        dst_v, mz_v, sg = bufs
        base = wid * EDGES_PER_TILE + b * CB
        pltpu.sync_copy(ei_hbm.at[1, pl.ds(base, CB)], dst_v)
        pltpu.async_copy(mz_hbm.at[pl.ds(base, CB)], mz_v, sg)

    def wait_load(bufs):
        _, mz_v, sg = bufs
        pltpu.make_async_copy(mz_hbm.at[pl.ds(0, CB)], mz_v, sg).wait()

    def add(bufs):
        dst_v, mz_v, _ = bufs
        pltpu.sync_copy(mz_v, mz_sh.at[dst_v], add=True)

    issue_load(0, s0)

    def _pair(i, c):
        b0 = i * 2
        issue_load(b0 + 1, s1)
        wait_load(s0)
        add(s0)
        issue_load(b0 + 2, s0)
        wait_load(s1)
        add(s1)
        return c

    lax.fori_loop(0, (NCHUNK - 1) // 2, _pair, 0)
    wait_load(s0)
    add(s0)
    plsc.subcore_barrier()

    pltpu.sync_copy(mz_sh.at[pl.ds(nbase, NODE_ROWS_PER_TILE)],
                    mz_parts_hbm.at[cid, pl.ds(nbase, NODE_ROWS_PER_TILE)])


_pass2_kernel = pl.kernel(
    _pass2_body,
    out_type=jax.ShapeDtypeStruct((NC, NODE_PAD, MZ), jnp.float32),
    mesh=plsc.VectorSubcoreMesh(core_axis_name="c", subcore_axis_name="s",
                                num_cores=NC, num_subcores=NS),
    compiler_params=pltpu.CompilerParams(use_tc_tiling_on_sc=False,
                                         needs_layout_passes=False),
    scratch_types=[
        pltpu.VMEM((CB,), jnp.int32),
        pltpu.VMEM((CB, MZ), jnp.float32),
        pltpu.VMEM((CB,), jnp.int32),
        pltpu.VMEM((CB, MZ), jnp.float32),
        pltpu.VMEM_SHARED((NODE_PAD, MZ), jnp.float32),
        pltpu.SemaphoreType.DMA,
        pltpu.SemaphoreType.DMA,
    ],
)


def _finalize_body(mz_ref, o_ref):
    mz = mz_ref[0] + mz_ref[1]
    wv = mz[:, 0:HD]
    z8 = mz[:, HD:HD + NUM_HEADS]
    row = lax.broadcasted_iota(jnp.int32, (NUM_HEADS, HD), 0)
    col = lax.broadcasted_iota(jnp.int32, (NUM_HEADS, HD), 1)
    expand = jnp.where(col // OUT_DIM == row, 1.0, 0.0)
    zrep = jnp.dot(z8, expand, preferred_element_type=jnp.float32)
    o_ref[...] = wv / (zrep + 1e-6)


def _finalize(mz_parts, block_rows=1024):
    return pl.pallas_call(
        _finalize_body,
        grid=(NODE_PAD // block_rows,),
        in_specs=[pl.BlockSpec((NC, block_rows, MZ), lambda i: (0, i, 0))],
        out_specs=pl.BlockSpec((block_rows, HD), lambda i: (i, 0)),
        out_shape=jax.ShapeDtypeStruct((NODE_PAD, HD), jnp.float32),
    )(mz_parts)


def kernel(h, e, edge_index, WQ, WK, WV, We):
    q_h = _matmul(h, WQ, 1000)
    kv = _matmul(h, jnp.concatenate([WK, WV], axis=1), 1000)
    p = _matmul(e, We * (1.0 / jnp.sqrt(jnp.float32(OUT_DIM))), 3200)

    zmz = jnp.zeros((NODE_ROWS_PER_TILE, MZ), jnp.float32)

    e_out, mz = _pass1_kernel(q_h, kv, p, edge_index)
    mz_parts = _pass2_kernel(mz, edge_index, zmz)
    h_out = _finalize(mz_parts)

    return (h_out[:N_NODES].reshape(N_NODES, NUM_HEADS, OUT_DIM),
            e_out.reshape(N_EDGES, NUM_HEADS, OUT_DIM))


# speedup vs baseline: 2.5081x; 1.1393x over previous
import jax
import jax.numpy as jnp
from jax import lax
from jax.experimental import pallas as pl
from jax.experimental.pallas import tpu as pltpu
from jax.experimental.pallas import tpu_sc as plsc

N_NODES = 10000
N_EDGES = 320000
IN_DIM = 128
NUM_HEADS = 8
OUT_DIM = 16
HD = NUM_HEADS * OUT_DIM
MZ = HD + 16

NC = 2
NS = 16
L = 16
NW = NC * NS
EDGES_PER_TILE = N_EDGES // NW
EB = 80
NBATCH = EDGES_PER_TILE // EB
CB = 80
NCHUNK = EDGES_PER_TILE // CB
NODE_PAD = 10240
NODE_ROWS_PER_TILE = NODE_PAD // NS


def _mm_body(x_ref, w_ref, o_ref):
    o_ref[...] = jnp.dot(x_ref[...], w_ref[...],
                         preferred_element_type=jnp.float32)


def _matmul(x, w, block_rows):
    m, k = x.shape
    n = w.shape[1]
    return pl.pallas_call(
        _mm_body,
        grid=(m // block_rows,),
        in_specs=[pl.BlockSpec((block_rows, k), lambda i: (i, 0)),
                  pl.BlockSpec((k, n), lambda i: (0, 0))],
        out_specs=pl.BlockSpec((block_rows, n), lambda i: (i, 0)),
        out_shape=jax.ShapeDtypeStruct((m, n), jnp.float32),
    )(x, w)


def _pass1_body(q_hbm, kv_hbm, p_hbm, ei_hbm,
                eout_hbm, mz_hbm,
                src0, dst0, kv0, q0, p0, eout0, mz0,
                src1, dst1, kv1, q1, p1, eout1, mz1,
                sg0, sg1, sw0, sw1):
    cid = lax.axis_index("c")
    sid = lax.axis_index("s")
    wid = sid * NC + cid

    zero16 = jnp.zeros((L,), jnp.float32)
    iota = lax.iota(jnp.int32, L)

    set0 = (src0, dst0, kv0, q0, p0, sg0)
    set1 = (src1, dst1, kv1, q1, p1, sg1)
    out0 = (eout0, mz0, sw0)
    out1 = (eout1, mz1, sw1)

    def issue_gathers(b, bufs):
        src_v, dst_v, kv_v, q_v, p_v, sg = bufs
        base = wid * EDGES_PER_TILE + b * EB
        pltpu.sync_copy(ei_hbm.at[0, pl.ds(base, EB)], src_v)
        pltpu.sync_copy(ei_hbm.at[1, pl.ds(base, EB)], dst_v)
        pltpu.async_copy(kv_hbm.at[src_v], kv_v, sg)
        pltpu.async_copy(q_hbm.at[dst_v], q_v, sg)
        pltpu.async_copy(p_hbm.at[pl.ds(base, EB)], p_v, sg)

    def wait_gathers(bufs):
        _, _, kv_v, q_v, p_v, sg = bufs
        pltpu.make_async_copy(kv_hbm.at[pl.ds(0, EB)], kv_v, sg).wait()
        pltpu.make_async_copy(q_hbm.at[pl.ds(0, EB)], q_v, sg).wait()
        pltpu.make_async_copy(p_hbm.at[pl.ds(0, EB)], p_v, sg).wait()

    def issue_wb(b, outs):
        eout_v, mz_v, sw = outs
        base = wid * EDGES_PER_TILE + b * EB
        pltpu.async_copy(eout_v, eout_hbm.at[pl.ds(base, EB)], sw)
        pltpu.async_copy(mz_v, mz_hbm.at[pl.ds(base, EB)], sw)

    def wait_wb(outs):
        eout_v, mz_v, sw = outs
        pltpu.make_async_copy(eout_v, eout_hbm.at[pl.ds(0, EB)], sw).wait()
        pltpu.make_async_copy(mz_v, mz_hbm.at[pl.ds(0, EB)], sw).wait()

    IL = 8

    def compute(bufs, outs):
        _, _, kv_v, q_v, p_v, _ = bufs
        eout_v, mz_v, _ = outs

        def _edge_quad(i, c):
            es = [i * IL + k for k in range(IL)]
            zaccs = [zero16] * IL
            for hh in range(NUM_HEADS):
                sl = pl.ds(hh * OUT_DIM, OUT_DIM)
                slv = pl.ds(HD + hh * OUT_DIM, OUT_DIM)
                scs = [kv_v[e, sl] * q_v[e, sl] * p_v[e, sl] for e in es]
                for k, e in enumerate(es):
                    eout_v[e, sl] = scs[k]
                avs = [jnp.exp(jnp.clip(jnp.full((L,), jnp.sum(sc),
                                                 jnp.float32),
                                        -5.0, 5.0)) for sc in scs]
                for k, e in enumerate(es):
                    mz_v[e, sl] = kv_v[e, slv] * avs[k]
                    zaccs[k] = jnp.where(iota == hh, avs[k], zaccs[k])
            for k, e in enumerate(es):
                mz_v[e, pl.ds(HD, 16)] = zaccs[k]
            return c

        lax.fori_loop(0, EB // IL, _edge_quad, 0)

    issue_gathers(0, set0)

    def _pair(i, c):
        b0 = i * 2

        issue_gathers(b0 + 1, set1)
        wait_gathers(set0)
        pl.when(i > 0)(lambda: wait_wb(out0))
        compute(set0, out0)
        issue_wb(b0, out0)

        issue_gathers(b0 + 2, set0)
        wait_gathers(set1)
        pl.when(i > 0)(lambda: wait_wb(out1))
        compute(set1, out1)
        issue_wb(b0 + 1, out1)
        return c

    lax.fori_loop(0, (NBATCH - 1) // 2, _pair, 0)

    wait_gathers(set0)
    wait_wb(out0)
    compute(set0, out0)
    issue_wb(NBATCH - 1, out0)
    wait_wb(out0)
    wait_wb(out1)


_pass1_kernel = pl.kernel(
    _pass1_body,
    out_type=(jax.ShapeDtypeStruct((N_EDGES, HD), jnp.float32),
              jax.ShapeDtypeStruct((N_EDGES, MZ), jnp.float32)),
    mesh=plsc.VectorSubcoreMesh(core_axis_name="c", subcore_axis_name="s",
                                num_cores=NC, num_subcores=NS),
    compiler_params=pltpu.CompilerParams(use_tc_tiling_on_sc=False,
                                         needs_layout_passes=False),
    scratch_types=[
        pltpu.VMEM((EB,), jnp.int32),
        pltpu.VMEM((EB,), jnp.int32),
        pltpu.VMEM((EB, 2 * HD), jnp.float32),
        pltpu.VMEM((EB, HD), jnp.float32),
        pltpu.VMEM((EB, HD), jnp.float32),
        pltpu.VMEM((EB, HD), jnp.float32),
        pltpu.VMEM((EB, MZ), jnp.float32),
        pltpu.VMEM((EB,), jnp.int32),
        pltpu.VMEM((EB,), jnp.int32),
        pltpu.VMEM((EB, 2 * HD), jnp.float32),
        pltpu.VMEM((EB, HD), jnp.float32),
        pltpu.VMEM((EB, HD), jnp.float32),
        pltpu.VMEM((EB, HD), jnp.float32),
        pltpu.VMEM((EB, MZ), jnp.float32),
        pltpu.SemaphoreType.DMA,
        pltpu.SemaphoreType.DMA,
        pltpu.SemaphoreType.DMA,
        pltpu.SemaphoreType.DMA,
    ],
)


def _pass2_body(mz_hbm, ei_hbm, zmz_hbm,
                mz_parts_hbm,
                dst0, mz0, dst1, mz1, mz_sh, sg0, sg1):
    cid = lax.axis_index("c")
    sid = lax.axis_index("s")
    wid = sid * NC + cid

    nbase = sid * NODE_ROWS_PER_TILE
    pltpu.sync_copy(zmz_hbm, mz_sh.at[pl.ds(nbase, NODE_ROWS_PER_TILE)])
    plsc.subcore_barrier()

    s0 = (dst0, mz0, sg0)
    s1 = (dst1, mz1, sg1)

    def issue_load(b, bufs):
        dst_v, mz_v, sg = bufs
        base = wid * EDGES_PER_TILE + b * CB
        pltpu.sync_copy(ei_hbm.at[1, pl.ds(base, CB)], dst_v)
        pltpu.async_copy(mz_hbm.at[pl.ds(base, CB)], mz_v, sg)

    def wait_load(bufs):
        _, mz_v, sg = bufs
        pltpu.make_async_copy(mz_hbm.at[pl.ds(0, CB)], mz_v, sg).wait()

    def add(bufs):
        dst_v, mz_v, _ = bufs
        pltpu.sync_copy(mz_v, mz_sh.at[dst_v], add=True)

    issue_load(0, s0)

    def _pair(i, c):
        b0 = i * 2
        issue_load(b0 + 1, s1)
        wait_load(s0)
        add(s0)
        issue_load(b0 + 2, s0)
        wait_load(s1)
        add(s1)
        return c

    lax.fori_loop(0, (NCHUNK - 1) // 2, _pair, 0)
    wait_load(s0)
    add(s0)
    plsc.subcore_barrier()

    pltpu.sync_copy(mz_sh.at[pl.ds(nbase, NODE_ROWS_PER_TILE)],
                    mz_parts_hbm.at[cid, pl.ds(nbase, NODE_ROWS_PER_TILE)])


_pass2_kernel = pl.kernel(
    _pass2_body,
    out_type=jax.ShapeDtypeStruct((NC, NODE_PAD, MZ), jnp.float32),
    mesh=plsc.VectorSubcoreMesh(core_axis_name="c", subcore_axis_name="s",
                                num_cores=NC, num_subcores=NS),
    compiler_params=pltpu.CompilerParams(use_tc_tiling_on_sc=False,
                                         needs_layout_passes=False),
    scratch_types=[
        pltpu.VMEM((CB,), jnp.int32),
        pltpu.VMEM((CB, MZ), jnp.float32),
        pltpu.VMEM((CB,), jnp.int32),
        pltpu.VMEM((CB, MZ), jnp.float32),
        pltpu.VMEM_SHARED((NODE_PAD, MZ), jnp.float32),
        pltpu.SemaphoreType.DMA,
        pltpu.SemaphoreType.DMA,
    ],
)


def _finalize_body(mz_ref, o_ref):
    mz = mz_ref[0] + mz_ref[1]
    wv = mz[:, 0:HD]
    z8 = mz[:, HD:HD + NUM_HEADS]
    row = lax.broadcasted_iota(jnp.int32, (NUM_HEADS, HD), 0)
    col = lax.broadcasted_iota(jnp.int32, (NUM_HEADS, HD), 1)
    expand = jnp.where(col // OUT_DIM == row, 1.0, 0.0)
    zrep = jnp.dot(z8, expand, preferred_element_type=jnp.float32)
    o_ref[...] = wv / (zrep + 1e-6)


def _finalize(mz_parts, block_rows=1024):
    return pl.pallas_call(
        _finalize_body,
        grid=(NODE_PAD // block_rows,),
        in_specs=[pl.BlockSpec((NC, block_rows, MZ), lambda i: (0, i, 0))],
        out_specs=pl.BlockSpec((block_rows, HD), lambda i: (i, 0)),
        out_shape=jax.ShapeDtypeStruct((NODE_PAD, HD), jnp.float32),
    )(mz_parts)


def kernel(h, e, edge_index, WQ, WK, WV, We):
    q_h = _matmul(h, WQ, 1000)
    kv = _matmul(h, jnp.concatenate([WK, WV], axis=1), 1000)
    p = _matmul(e, We * (1.0 / jnp.sqrt(jnp.float32(OUT_DIM))), 3200)

    zmz = jnp.zeros((NODE_ROWS_PER_TILE, MZ), jnp.float32)

    e_out, mz = _pass1_kernel(q_h, kv, p, edge_index)
    mz_parts = _pass2_kernel(mz, edge_index, zmz)
    h_out = _finalize(mz_parts)

    return (h_out[:N_NODES].reshape(N_NODES, NUM_HEADS, OUT_DIM),
            e_out.reshape(N_EDGES, NUM_HEADS, OUT_DIM))
